# baseline (speedup 1.0000x reference)
import functools

import jax
import jax.numpy as jnp
from jax import lax
from jax.experimental import pallas as pl
from jax.experimental.pallas import tpu as pltpu
from jax.experimental.pallas import tpu_sc as plsc

N = 10000
E = 320000
H = 128
HE = 256

NC = 2
NS = 16
NW = NC * NS
B = 80
UNR = 5
UNS = 2
N2 = 10240
RPS = N2 // NS

CHUNKS = ((0, 153600), (153600, 166400))
BS = 128

RB = 1000
NRB = N // RB
EB = 2560


def _mesh():
    return plsc.VectorSubcoreMesh(
        core_axis_name="c", subcore_axis_name="s", num_cores=NC, num_subcores=NS)


def _gather_chunk(e_off, e_cnt):
    ept = e_cnt // NW
    nfull = ept // BS
    rem = ept % BS
    nouter = nfull // UNR
    nodd = nfull % UNR
    assert rem == 0 or nodd < UNR

    def body(x_hbm, src_hbm, xg_hbm, srcv, *bufs):
        rows = bufs[0:UNR]
        gsem = bufs[UNR:2 * UNR]
        wsem = bufs[2 * UNR:3 * UNR]
        w = lax.axis_index("c") * NS + lax.axis_index("s")
        base_l = w * ept
        pltpu.sync_copy(src_hbm.at[pl.ds(e_off + base_l, ept)], srcv)

        def outer(it, carry):
            base = it * UNR
            gds = []
            for b in range(UNR):
                blk = base + b
                gds.append(pltpu.async_copy(
                    x_hbm.at[srcv.at[pl.ds(blk * BS, BS)]], rows[b], gsem[b]))
            wds = []
            for b in range(UNR):
                blk = base + b
                gds[b].wait()
                wds.append(pltpu.async_copy(
                    rows[b], xg_hbm.at[pl.ds(base_l + blk * BS, BS)],
                    wsem[b]))
            for b in range(UNR):
                wds[b].wait()
            return carry

        lax.fori_loop(0, nouter, outer, 0)
        gds = []
        for k in range(nodd):
            blk = nouter * UNR + k
            gds.append(pltpu.async_copy(
                x_hbm.at[srcv.at[pl.ds(blk * BS, BS)]], rows[k], gsem[k]))
        grem = None
        if rem:
            grem = pltpu.async_copy(
                x_hbm.at[srcv.at[pl.ds(nfull * BS, rem)]],
                rows[nodd].at[pl.ds(0, rem)], gsem[nodd])
        wds = []
        for k in range(nodd):
            blk = nouter * UNR + k
            gds[k].wait()
            wds.append(pltpu.async_copy(
                rows[k], xg_hbm.at[pl.ds(base_l + blk * BS, BS)], wsem[k]))
        if rem:
            grem.wait()
            wds.append(pltpu.async_copy(
                rows[nodd].at[pl.ds(0, rem)],
                xg_hbm.at[pl.ds(base_l + nfull * BS, rem)], wsem[nodd]))
        for d in wds:
            d.wait()

    return functools.partial(
        pl.kernel,
        out_type=jax.ShapeDtypeStruct((e_cnt, H), jnp.float32),
        mesh=_mesh(),
        scratch_types=(
            [pltpu.VMEM((ept,), jnp.int32)]
            + [pltpu.VMEM((BS, H), jnp.float32)] * UNR
            + [pltpu.SemaphoreType.DMA] * (2 * UNR)
        ),
    )(body)


def _v_body(xg_ref, attrt_ref, bases_ref, we_ref, be_ref, wp1_ref, bp1_ref,
            out_ref):
    e = lax.dot_general(attrt_ref[...], we_ref[...],
                        (((0,), (0,)), ((), ()))) + be_ref[...]
    pos = xg_ref[...] + e
    out_ref[...] = jnp.maximum(pos @ wp1_ref[...] + bp1_ref[...], 0.0) \
        * bases_ref[...]


def _make_v(xg, attrt, bases, W_e, b_e, W_p1, b_p1, e_off, e_cnt):
    neb = e_cnt // EB
    offb = e_off // EB
    return pl.pallas_call(
        _v_body,
        grid=(neb,),
        in_specs=[
            pl.BlockSpec((EB, H), lambda i: (i, 0)),
            pl.BlockSpec((7, EB), lambda i: (0, offb + i)),
            pl.BlockSpec((EB, HE), lambda i: (offb + i, 0)),
            pl.BlockSpec((7, H), lambda i: (0, 0)),
            pl.BlockSpec((1, H), lambda i: (0, 0)),
            pl.BlockSpec((H, HE), lambda i: (0, 0)),
            pl.BlockSpec((1, HE), lambda i: (0, 0)),
        ],
        out_specs=pl.BlockSpec((EB, HE), lambda i: (i, 0)),
        out_shape=jax.ShapeDtypeStruct((e_cnt, HE), jnp.float32),
    )(xg, attrt, bases, W_e, b_e.reshape(1, H), W_p1, b_p1.reshape(1, HE))


def _scatter_chunk(e_off, e_cnt):
    eps = e_cnt // NS
    nfull = eps // BS
    rem = eps % BS
    npair = nfull // UNS
    nodd = nfull % UNS

    def body(v_hbm, dst_hbm, zeros_hbm, out_hbm, *bufs):
        dbufs = bufs[0:UNS]
        vbufs = bufs[UNS:2 * UNS]
        isem = bufs[2 * UNS:3 * UNS]
        vsem = bufs[3 * UNS:4 * UNS]
        asem = bufs[4 * UNS:5 * UNS]
        rest = bufs[5 * UNS:]
        if rem:
            dbr, vbr = rest[0], rest[1]
            aggr_sp = rest[2]
        else:
            aggr_sp = rest[0]
        c = lax.axis_index("c")
        s = lax.axis_index("s")
        row0 = s * RPS
        pltpu.sync_copy(zeros_hbm, aggr_sp.at[pl.ds(row0, RPS)])
        plsc.subcore_barrier()

        base_l = s * eps
        col0 = c * H

        def outer(it, carry):
            base = it * UNS
            ids, vds = [], []
            for b in range(UNS):
                e0 = base_l + (base + b) * BS
                ids.append(pltpu.async_copy(
                    dst_hbm.at[pl.ds(e_off + e0, BS)], dbufs[b], isem[b]))
                vds.append(pltpu.async_copy(
                    v_hbm.at[pl.ds(e0, BS), pl.ds(col0, H)], vbufs[b],
                    vsem[b]))
            ads = []
            for b in range(UNS):
                ids[b].wait()
                vds[b].wait()
                ads.append(pltpu.async_copy(
                    vbufs[b], aggr_sp.at[dbufs[b]], asem[b], add=True))
            for b in range(UNS):
                ads[b].wait()
            return carry

        lax.fori_loop(0, npair, outer, 0)
        for k in range(nodd):
            e0 = base_l + (npair * UNS + k) * BS
            icp = pltpu.async_copy(
                dst_hbm.at[pl.ds(e_off + e0, BS)], dbufs[0], isem[0])
            vcp = pltpu.async_copy(
                v_hbm.at[pl.ds(e0, BS), pl.ds(col0, H)], vbufs[0], vsem[0])
            icp.wait()
            vcp.wait()
            pltpu.sync_copy(vbufs[0], aggr_sp.at[dbufs[0]], add=True)
        if rem:
            e0 = base_l + nfull * BS
            icp = pltpu.async_copy(
                dst_hbm.at[pl.ds(e_off + e0, rem)], dbr, isem[0])
            vcp = pltpu.async_copy(
                v_hbm.at[pl.ds(e0, rem), pl.ds(col0, H)], vbr, vsem[0])
            icp.wait()
            vcp.wait()
            pltpu.sync_copy(vbr, aggr_sp.at[dbr], add=True)
        plsc.subcore_barrier()
        pltpu.sync_copy(aggr_sp.at[pl.ds(row0, RPS)],
                        out_hbm.at[c, pl.ds(row0, RPS)])

    return functools.partial(
        pl.kernel,
        out_type=jax.ShapeDtypeStruct((NC, N2, H), jnp.float32),
        mesh=_mesh(),
        scratch_types=(
            [pltpu.VMEM((BS,), jnp.int32)] * UNS
            + [pltpu.VMEM((BS, H), jnp.float32)] * UNS
            + [pltpu.SemaphoreType.DMA] * (3 * UNS)
            + ([pltpu.VMEM((rem,), jnp.int32),
                pltpu.VMEM((rem, H), jnp.float32)] if rem else [])
            + [pltpu.VMEM_SHARED((N2, H), jnp.float32)]
        ),
    )(body)


def _ffn_body(a1lo_ref, a1hi_ref, a2lo_ref, a2hi_ref,
              x_ref, wp2_ref, bp2_ref, wf1_ref, bf1_ref, wf2_ref, bf2_ref,
              h2_ref, xtx_ref, sh_ref, ssq_ref):
    i = pl.program_id(0)
    aggr = jnp.concatenate([a1lo_ref[0] + a2lo_ref[0],
                            a1hi_ref[0] + a2hi_ref[0]],
                           axis=1)
    y = aggr + jnp.maximum(x_ref[...] @ wp2_ref[...] + bp2_ref[...], 0.0)

    @pl.when(i == 0)
    def _():
        xtx_ref[...] = jnp.zeros_like(xtx_ref)
        sh_ref[...] = jnp.zeros_like(sh_ref)
        ssq_ref[...] = jnp.zeros_like(ssq_ref)

    xtx_ref[...] += lax.dot_general(y, y, (((0,), (0,)), ((), ())))
    h = jnp.maximum(y @ wf1_ref[...] + bf1_ref[...], 0.0)
    h2 = jnp.maximum(h @ wf2_ref[...] + bf2_ref[...], 0.0)
    h2_ref[...] = h2
    sh_ref[...] += jnp.sum(h2, axis=0, keepdims=True)
    ssq_ref[...] += jnp.sum(h2 * h2, axis=0, keepdims=True)


def _make_ffn(ag1, ag2, x_feat, W_p2, b_p2, W_f1, b_f1, W_f2, b_f2):
    aspec = [
        pl.BlockSpec((1, RB, H), lambda i: (0, i, 0)),
        pl.BlockSpec((1, RB, H), lambda i: (1, i, 0)),
    ]
    return pl.pallas_call(
        _ffn_body,
        grid=(NRB,),
        in_specs=aspec + aspec + [
            pl.BlockSpec((RB, H), lambda i: (i, 0)),
            pl.BlockSpec((H, HE), lambda i: (0, 0)),
            pl.BlockSpec((1, HE), lambda i: (0, 0)),
            pl.BlockSpec((HE, HE), lambda i: (0, 0)),
            pl.BlockSpec((1, HE), lambda i: (0, 0)),
            pl.BlockSpec((HE, H), lambda i: (0, 0)),
            pl.BlockSpec((1, H), lambda i: (0, 0)),
        ],
        out_specs=[
            pl.BlockSpec((RB, H), lambda i: (i, 0)),
            pl.BlockSpec((HE, HE), lambda i: (0, 0)),
            pl.BlockSpec((1, H), lambda i: (0, 0)),
            pl.BlockSpec((1, H), lambda i: (0, 0)),
        ],
        out_shape=[
            jax.ShapeDtypeStruct((N, H), jnp.float32),
            jax.ShapeDtypeStruct((HE, HE), jnp.float32),
            jax.ShapeDtypeStruct((1, H), jnp.float32),
            jax.ShapeDtypeStruct((1, H), jnp.float32),
        ],
    )(ag1, ag1, ag2, ag2, x_feat, W_p2, b_p2.reshape(1, HE), W_f1,
      b_f1.reshape(1, HE), W_f2, b_f2.reshape(1, H))


def _bn_body(h2_ref, sh_ref, ssq_ref, g_ref, b_ref, out_ref):
    mean = sh_ref[...] / N
    var = ssq_ref[...] / N - mean * mean
    rstd = lax.rsqrt(var + 1e-5)
    out_ref[...] = (h2_ref[...] - mean) * (rstd * g_ref[...]) + b_ref[...]


def _make_bn(h2, sh, ssq, gamma, beta):
    return pl.pallas_call(
        _bn_body,
        grid=(NRB,),
        in_specs=[
            pl.BlockSpec((RB, H), lambda i: (i, 0)),
            pl.BlockSpec((1, H), lambda i: (0, 0)),
            pl.BlockSpec((1, H), lambda i: (0, 0)),
            pl.BlockSpec((1, H), lambda i: (0, 0)),
            pl.BlockSpec((1, H), lambda i: (0, 0)),
        ],
        out_specs=pl.BlockSpec((RB, H), lambda i: (i, 0)),
        out_shape=jax.ShapeDtypeStruct((N, H), jnp.float32),
    )(h2, sh, ssq, gamma.reshape(1, H), beta.reshape(1, H))


def kernel(x_feat, edge_index, edge_attr, bases, W_e, b_e, W_p1, b_p1,
           W_p2, b_p2, W_f1, b_f1, W_f2, b_f2, gamma, beta):
    src = edge_index[0]
    dst = edge_index[1]
    attrt = edge_attr.T
    zeros = jnp.zeros((RPS, H), jnp.float32)

    ags = []
    for e_off, e_cnt in CHUNKS:
        xg = _gather_chunk(e_off, e_cnt)(x_feat, src)
        v = _make_v(xg, attrt, bases, W_e, b_e, W_p1, b_p1, e_off, e_cnt)
        ags.append(_scatter_chunk(e_off, e_cnt)(v, dst, zeros))

    h2, xtx, sh, ssq = _make_ffn(ags[0], ags[1], x_feat, W_p2, b_p2,
                                 W_f1, b_f1, W_f2, b_f2)
    out = _make_bn(h2, sh, ssq, gamma, beta)
    return (out, xtx)


# speedup vs baseline: 1.0025x; 1.0025x over previous
import functools

import jax
import jax.numpy as jnp
from jax import lax
from jax.experimental import pallas as pl
from jax.experimental.pallas import tpu as pltpu
from jax.experimental.pallas import tpu_sc as plsc

N = 10000
E = 320000
H = 128
HE = 256

NC = 2
NS = 16
NW = NC * NS
B = 80
UNR = 5
UNS = 2
N2 = 10240
RPS = N2 // NS

CHUNKS = ((0, 153600), (153600, 166400))
BS = 128

RB = 1000
NRB = N // RB
EB = 2560


def _mesh():
    return plsc.VectorSubcoreMesh(
        core_axis_name="c", subcore_axis_name="s", num_cores=NC, num_subcores=NS)


def _gather_chunk(e_off, e_cnt):
    ept = e_cnt // NW
    gblk = ept // B
    assert gblk % UNR == 0

    def body(x_hbm, src_hbm, xg_hbm, srcv, *bufs):
        rows = bufs[0:UNR]
        gsem = bufs[UNR:2 * UNR]
        wsem = bufs[2 * UNR:3 * UNR]
        w = lax.axis_index("c") * NS + lax.axis_index("s")
        base_l = w * ept
        pltpu.sync_copy(src_hbm.at[pl.ds(e_off + base_l, ept)], srcv)

        def outer(it, carry):
            base = it * UNR
            gds = []
            for b in range(UNR):
                blk = base + b
                gds.append(pltpu.async_copy(
                    x_hbm.at[srcv.at[pl.ds(blk * B, B)]], rows[b], gsem[b]))
            wds = []
            for b in range(UNR):
                blk = base + b
                gds[b].wait()
                wds.append(pltpu.async_copy(
                    rows[b], xg_hbm.at[pl.ds(base_l + blk * B, B)], wsem[b]))
            for b in range(UNR):
                wds[b].wait()
            return carry

        lax.fori_loop(0, gblk // UNR, outer, 0)

    return functools.partial(
        pl.kernel,
        out_type=jax.ShapeDtypeStruct((e_cnt, H), jnp.float32),
        mesh=_mesh(),
        scratch_types=(
            [pltpu.VMEM((ept,), jnp.int32)]
            + [pltpu.VMEM((B, H), jnp.float32)] * UNR
            + [pltpu.SemaphoreType.DMA] * (2 * UNR)
        ),
    )(body)


def _v_body(xg_ref, attrt_ref, bases_ref, we_ref, be_ref, wp1_ref, bp1_ref,
            out_ref):
    e = lax.dot_general(attrt_ref[...], we_ref[...],
                        (((0,), (0,)), ((), ()))) + be_ref[...]
    pos = xg_ref[...] + e
    out_ref[...] = jnp.maximum(pos @ wp1_ref[...] + bp1_ref[...], 0.0) \
        * bases_ref[...]


def _make_v(xg, attrt, bases, W_e, b_e, W_p1, b_p1, e_off, e_cnt):
    neb = e_cnt // EB
    offb = e_off // EB
    return pl.pallas_call(
        _v_body,
        grid=(neb,),
        in_specs=[
            pl.BlockSpec((EB, H), lambda i: (i, 0)),
            pl.BlockSpec((7, EB), lambda i: (0, offb + i)),
            pl.BlockSpec((EB, HE), lambda i: (offb + i, 0)),
            pl.BlockSpec((7, H), lambda i: (0, 0)),
            pl.BlockSpec((1, H), lambda i: (0, 0)),
            pl.BlockSpec((H, HE), lambda i: (0, 0)),
            pl.BlockSpec((1, HE), lambda i: (0, 0)),
        ],
        out_specs=pl.BlockSpec((EB, HE), lambda i: (i, 0)),
        out_shape=jax.ShapeDtypeStruct((e_cnt, HE), jnp.float32),
    )(xg, attrt, bases, W_e, b_e.reshape(1, H), W_p1, b_p1.reshape(1, HE))


def _scatter_chunk(e_off, e_cnt):
    eps = e_cnt // NS
    nfull = eps // BS
    rem = eps % BS
    npair = nfull // UNS
    nodd = nfull % UNS

    def body(v_hbm, dst_hbm, zeros_hbm, out_hbm, *bufs):
        dbufs = bufs[0:UNS]
        vbufs = bufs[UNS:2 * UNS]
        isem = bufs[2 * UNS:3 * UNS]
        vsem = bufs[3 * UNS:4 * UNS]
        asem = bufs[4 * UNS:5 * UNS]
        rest = bufs[5 * UNS:]
        if rem:
            dbr, vbr = rest[0], rest[1]
            aggr_sp = rest[2]
        else:
            aggr_sp = rest[0]
        c = lax.axis_index("c")
        s = lax.axis_index("s")
        row0 = s * RPS
        pltpu.sync_copy(zeros_hbm, aggr_sp.at[pl.ds(row0, RPS)])
        plsc.subcore_barrier()

        base_l = s * eps
        col0 = c * H

        def outer(it, carry):
            base = it * UNS
            ids, vds = [], []
            for b in range(UNS):
                e0 = base_l + (base + b) * BS
                ids.append(pltpu.async_copy(
                    dst_hbm.at[pl.ds(e_off + e0, BS)], dbufs[b], isem[b]))
                vds.append(pltpu.async_copy(
                    v_hbm.at[pl.ds(e0, BS), pl.ds(col0, H)], vbufs[b],
                    vsem[b]))
            ads = []
            for b in range(UNS):
                ids[b].wait()
                vds[b].wait()
                ads.append(pltpu.async_copy(
                    vbufs[b], aggr_sp.at[dbufs[b]], asem[b], add=True))
            for b in range(UNS):
                ads[b].wait()
            return carry

        lax.fori_loop(0, npair, outer, 0)
        for k in range(nodd):
            e0 = base_l + (npair * UNS + k) * BS
            icp = pltpu.async_copy(
                dst_hbm.at[pl.ds(e_off + e0, BS)], dbufs[0], isem[0])
            vcp = pltpu.async_copy(
                v_hbm.at[pl.ds(e0, BS), pl.ds(col0, H)], vbufs[0], vsem[0])
            icp.wait()
            vcp.wait()
            pltpu.sync_copy(vbufs[0], aggr_sp.at[dbufs[0]], add=True)
        if rem:
            e0 = base_l + nfull * BS
            icp = pltpu.async_copy(
                dst_hbm.at[pl.ds(e_off + e0, rem)], dbr, isem[0])
            vcp = pltpu.async_copy(
                v_hbm.at[pl.ds(e0, rem), pl.ds(col0, H)], vbr, vsem[0])
            icp.wait()
            vcp.wait()
            pltpu.sync_copy(vbr, aggr_sp.at[dbr], add=True)
        plsc.subcore_barrier()
        pltpu.sync_copy(aggr_sp.at[pl.ds(row0, RPS)],
                        out_hbm.at[c, pl.ds(row0, RPS)])

    return functools.partial(
        pl.kernel,
        out_type=jax.ShapeDtypeStruct((NC, N2, H), jnp.float32),
        mesh=_mesh(),
        scratch_types=(
            [pltpu.VMEM((BS,), jnp.int32)] * UNS
            + [pltpu.VMEM((BS, H), jnp.float32)] * UNS
            + [pltpu.SemaphoreType.DMA] * (3 * UNS)
            + ([pltpu.VMEM((rem,), jnp.int32),
                pltpu.VMEM((rem, H), jnp.float32)] if rem else [])
            + [pltpu.VMEM_SHARED((N2, H), jnp.float32)]
        ),
    )(body)


def _ffn_body(a1lo_ref, a1hi_ref, a2lo_ref, a2hi_ref,
              x_ref, wp2_ref, bp2_ref, wf1_ref, bf1_ref, wf2_ref, bf2_ref,
              h2_ref, xtx_ref, sh_ref, ssq_ref):
    i = pl.program_id(0)
    aggr = jnp.concatenate([a1lo_ref[0] + a2lo_ref[0],
                            a1hi_ref[0] + a2hi_ref[0]],
                           axis=1)
    y = aggr + jnp.maximum(x_ref[...] @ wp2_ref[...] + bp2_ref[...], 0.0)

    @pl.when(i == 0)
    def _():
        xtx_ref[...] = jnp.zeros_like(xtx_ref)
        sh_ref[...] = jnp.zeros_like(sh_ref)
        ssq_ref[...] = jnp.zeros_like(ssq_ref)

    xtx_ref[...] += lax.dot_general(y, y, (((0,), (0,)), ((), ())))
    h = jnp.maximum(y @ wf1_ref[...] + bf1_ref[...], 0.0)
    h2 = jnp.maximum(h @ wf2_ref[...] + bf2_ref[...], 0.0)
    h2_ref[...] = h2
    sh_ref[...] += jnp.sum(h2, axis=0, keepdims=True)
    ssq_ref[...] += jnp.sum(h2 * h2, axis=0, keepdims=True)


def _make_ffn(ag1, ag2, x_feat, W_p2, b_p2, W_f1, b_f1, W_f2, b_f2):
    aspec = [
        pl.BlockSpec((1, RB, H), lambda i: (0, i, 0)),
        pl.BlockSpec((1, RB, H), lambda i: (1, i, 0)),
    ]
    return pl.pallas_call(
        _ffn_body,
        grid=(NRB,),
        in_specs=aspec + aspec + [
            pl.BlockSpec((RB, H), lambda i: (i, 0)),
            pl.BlockSpec((H, HE), lambda i: (0, 0)),
            pl.BlockSpec((1, HE), lambda i: (0, 0)),
            pl.BlockSpec((HE, HE), lambda i: (0, 0)),
            pl.BlockSpec((1, HE), lambda i: (0, 0)),
            pl.BlockSpec((HE, H), lambda i: (0, 0)),
            pl.BlockSpec((1, H), lambda i: (0, 0)),
        ],
        out_specs=[
            pl.BlockSpec((RB, H), lambda i: (i, 0)),
            pl.BlockSpec((HE, HE), lambda i: (0, 0)),
            pl.BlockSpec((1, H), lambda i: (0, 0)),
            pl.BlockSpec((1, H), lambda i: (0, 0)),
        ],
        out_shape=[
            jax.ShapeDtypeStruct((N, H), jnp.float32),
            jax.ShapeDtypeStruct((HE, HE), jnp.float32),
            jax.ShapeDtypeStruct((1, H), jnp.float32),
            jax.ShapeDtypeStruct((1, H), jnp.float32),
        ],
    )(ag1, ag1, ag2, ag2, x_feat, W_p2, b_p2.reshape(1, HE), W_f1,
      b_f1.reshape(1, HE), W_f2, b_f2.reshape(1, H))


def _bn_body(h2_ref, sh_ref, ssq_ref, g_ref, b_ref, out_ref):
    mean = sh_ref[...] / N
    var = ssq_ref[...] / N - mean * mean
    rstd = lax.rsqrt(var + 1e-5)
    out_ref[...] = (h2_ref[...] - mean) * (rstd * g_ref[...]) + b_ref[...]


def _make_bn(h2, sh, ssq, gamma, beta):
    return pl.pallas_call(
        _bn_body,
        grid=(NRB,),
        in_specs=[
            pl.BlockSpec((RB, H), lambda i: (i, 0)),
            pl.BlockSpec((1, H), lambda i: (0, 0)),
            pl.BlockSpec((1, H), lambda i: (0, 0)),
            pl.BlockSpec((1, H), lambda i: (0, 0)),
            pl.BlockSpec((1, H), lambda i: (0, 0)),
        ],
        out_specs=pl.BlockSpec((RB, H), lambda i: (i, 0)),
        out_shape=jax.ShapeDtypeStruct((N, H), jnp.float32),
    )(h2, sh, ssq, gamma.reshape(1, H), beta.reshape(1, H))


def kernel(x_feat, edge_index, edge_attr, bases, W_e, b_e, W_p1, b_p1,
           W_p2, b_p2, W_f1, b_f1, W_f2, b_f2, gamma, beta):
    src = edge_index[0]
    dst = edge_index[1]
    attrt = edge_attr.T
    zeros = jnp.zeros((RPS, H), jnp.float32)

    ags = []
    for e_off, e_cnt in CHUNKS:
        xg = _gather_chunk(e_off, e_cnt)(x_feat, src)
        v = _make_v(xg, attrt, bases, W_e, b_e, W_p1, b_p1, e_off, e_cnt)
        ags.append(_scatter_chunk(e_off, e_cnt)(v, dst, zeros))

    h2, xtx, sh, ssq = _make_ffn(ags[0], ags[1], x_feat, W_p2, b_p2,
                                 W_f1, b_f1, W_f2, b_f2)
    out = _make_bn(h2, sh, ssq, gamma, beta)
    return (out, xtx)
